# baseline (speedup 1.0000x reference)
import functools

import jax
import jax.numpy as jnp
from jax import lax
from jax.experimental import pallas as pl
from jax.experimental.pallas import tpu as pltpu
from jax.experimental.pallas import tpu_sc as plsc

EMB = 128
LANES = 16
ROWW = EMB // 2
OFF1, OFF2, OFF3 = 119, 191, 311
TROWS = 407


@functools.cache
def _launcher(n):
    nc, ns = 2, 16
    nw = nc * ns
    per_w = n // nw
    assert per_w * nw == n
    chunk = 125
    n_chunks = per_w // chunk
    n_groups = -(-per_w // LANES)
    per_w_pad = n_groups * LANES
    xi_words = 9 * per_w
    xi_size = 9 * per_w_pad + 16
    mesh = plsc.VectorSubcoreMesh(
        core_axis_name="c", subcore_axis_name="s", num_cores=nc, num_subcores=ns
    )

    @functools.partial(
        pl.kernel,
        mesh=mesh,
        compiler_params=pltpu.CompilerParams(needs_layout_passes=False),
        out_type=jax.ShapeDtypeStruct((n * EMB,), jnp.float32),
        scratch_types=[
            pltpu.VMEM((TROWS * ROWW,), jnp.int32),
            pltpu.VMEM((xi_size,), jnp.int32),
            pltpu.VMEM((4 * per_w_pad,), jnp.int32),
            pltpu.VMEM((chunk * EMB,), jnp.float32),
        ],
    )
    def launch(xi_hbm, t_hbm, out_hbm, t_v, xi_v, base_v, o_v0):
        wid = lax.axis_index("s") * nc + lax.axis_index("c")
        pltpu.sync_copy(t_hbm, t_v)
        off = wid * xi_words
        delta = lax.rem(off, 8)
        base = pl.multiple_of(off - delta, 8)
        pltpu.sync_copy(xi_hbm.at[pl.ds(base, xi_words + 16)],
                        xi_v.at[pl.ds(0, xi_words + 16)])
        bufs = (o_v0,)
        lanes = jax.lax.iota(jnp.int32, LANES)

        def base_body(g, _):
            av = (g * LANES + lanes) * 9 + delta
            xs = [plsc.load_gather(xi_v, [av + k]) for k in range(9)]
            b0 = xs[0] * ROWW
            b1 = (xs[2] * 6 + xs[5] + OFF1) * ROWW
            b2 = (xs[3] * 10 + xs[4] + OFF2) * ROWW
            b3 = (((xs[1] * 6 + xs[6]) * 2 + xs[7]) * 2 + xs[8] + OFF3) * ROWW
            ov = (g * LANES + lanes) * 4
            plsc.store_scatter(base_v, [ov], b0)
            plsc.store_scatter(base_v, [ov + 1], b1)
            plsc.store_scatter(base_v, [ov + 2], b2)
            plsc.store_scatter(base_v, [ov + 3], b3)
            return 0

        lax.fori_loop(0, n_groups, base_body, 0)

        def do_chunk(j, buf):
            ob = bufs[buf]

            def bfrow(r, c):
                return plsc.bitcast(t_v[pl.ds(r + c, LANES)], jnp.bfloat16)

            BLK = 5

            def blk_body(ib, _):
                i0 = ib * BLK
                packed = []
                for u in range(BLK):
                    a = j * chunk + i0 + u
                    bv = base_v[pl.ds(a * 4, LANES)]
                    r0, r1, r2, r3 = bv[0], bv[1], bv[2], bv[3]
                    packed.append([
                        (bfrow(r0, g * LANES) + bfrow(r1, g * LANES))
                        + (bfrow(r2, g * LANES) + bfrow(r3, g * LANES))
                        for g in range(EMB // (2 * LANES))
                    ])
                for u in range(BLK):
                    o = (i0 + u) * EMB
                    for g, v in enumerate(packed[u]):
                        va, vb = plsc.unpack(v, format=plsc.PackFormat.INTERLEAVED)
                        ob[pl.ds(o + g * 2 * LANES, LANES)] = va
                        ob[pl.ds(o + g * 2 * LANES + LANES, LANES)] = vb
                return 0

            lax.fori_loop(0, chunk // BLK, blk_body, 0)
            pltpu.sync_copy(
                ob,
                out_hbm.at[pl.ds((wid * per_w + j * chunk) * EMB, chunk * EMB)],
            )

        def chunk_loop(j, _):
            do_chunk(j, 0)
            return 0

        lax.fori_loop(0, n_chunks, chunk_loop, 0)

    return launch


def kernel(x, W0, W1, W2, W3, W4, W5, W6, W7, W8):
    n = x.shape[0]
    t1 = (W2[:, None, :] + W5[None, :, :]).reshape(-1, EMB)
    t2 = (W3[:, None, :] + W4[None, :, :]).reshape(-1, EMB)
    t3 = (
        W1[:, None, None, None, :]
        + W6[None, :, None, None, :]
        + W7[None, None, :, None, :]
        + W8[None, None, None, :, :]
    ).reshape(-1, EMB)
    t = jnp.concatenate([W0, t1, t2, t3], axis=0)
    t = t.reshape(-1, 4, 2, LANES).transpose(0, 1, 3, 2)
    t = t.astype(jnp.bfloat16).reshape(-1, 2)
    t = jax.lax.bitcast_convert_type(t, jnp.int32).reshape(-1)

    xi = x.astype(jnp.int32).reshape(-1)
    xi = jnp.pad(xi, (0, 16))
    out = _launcher(n)(xi, t)
    return out.reshape(n, EMB)


# speedup vs baseline: 1.0878x; 1.0878x over previous
import functools

import jax
import jax.numpy as jnp
from jax import lax
from jax.experimental import pallas as pl
from jax.experimental.pallas import tpu as pltpu
from jax.experimental.pallas import tpu_sc as plsc

EMB = 128
LANES = 16
ROWW = EMB // 2
OFF1, OFF2, OFF3 = 119, 191, 311
TROWS = 407
BLK = 5


@functools.cache
def _launcher(n):
    nc, ns = 2, 16
    nw = nc * ns
    per_w = (n // nw) // 8 * 8
    n_tail = n - per_w * nw
    tail_chunks = n_tail // 8
    assert tail_chunks <= nw
    chunk = 120
    n_chunks = per_w // chunk
    assert n_chunks % 2 == 0 and chunk % BLK == 0
    xi_main = 9 * per_w
    xi_tail = 144
    xi_size = xi_main + xi_tail
    n_groups = per_w // LANES
    base_n = per_w + LANES
    mesh = plsc.VectorSubcoreMesh(
        core_axis_name="c", subcore_axis_name="s", num_cores=nc, num_subcores=ns
    )

    @functools.partial(
        pl.kernel,
        mesh=mesh,
        compiler_params=pltpu.CompilerParams(needs_layout_passes=False),
        out_type=jax.ShapeDtypeStruct((n, EMB), jnp.float32),
        scratch_types=[
            pltpu.VMEM((TROWS * ROWW,), jnp.int32),
            pltpu.VMEM((xi_size,), jnp.int32),
            pltpu.VMEM((4 * base_n,), jnp.int32),
            pltpu.VMEM((chunk, EMB), jnp.float32),
            pltpu.VMEM((chunk, EMB), jnp.float32),
            pltpu.SemaphoreType.DMA,
            pltpu.SemaphoreType.DMA,
        ],
    )
    def launch(xi_hbm, t_hbm, out_hbm, t_v, xi_v, base_v, o_v0, o_v1, sem0, sem1):
        wid = lax.axis_index("s") * nc + lax.axis_index("c")
        pltpu.sync_copy(t_hbm, t_v)
        m_off = pl.multiple_of(wid * xi_main, 8)
        pltpu.sync_copy(xi_hbm.at[pl.ds(m_off, xi_main)],
                        xi_v.at[pl.ds(0, xi_main)])
        t_off = pl.multiple_of(
            per_w * nw * 9 + lax.min(wid, tail_chunks - 1) * 72, 8
        )
        pltpu.sync_copy(xi_hbm.at[pl.ds(t_off, xi_tail)],
                        xi_v.at[pl.ds(xi_main, xi_tail)])
        sems = (sem0, sem1)
        bufs = (o_v0, o_v1)
        lanes = jax.lax.iota(jnp.int32, LANES)

        def compute_bases(xi_base, atom0):
            av = xi_base + lanes * 9
            xs = [plsc.load_gather(xi_v, [av + k]) for k in range(9)]
            b0 = xs[0] * ROWW
            b1 = (xs[2] * 6 + xs[5] + OFF1) * ROWW
            b2 = (xs[3] * 10 + xs[4] + OFF2) * ROWW
            b3 = (((xs[1] * 6 + xs[6]) * 2 + xs[7]) * 2 + xs[8] + OFF3) * ROWW
            ov = (atom0 + lanes) * 4
            plsc.store_scatter(base_v, [ov], b0)
            plsc.store_scatter(base_v, [ov + 1], b1)
            plsc.store_scatter(base_v, [ov + 2], b2)
            plsc.store_scatter(base_v, [ov + 3], b3)

        def base_body(g, _):
            compute_bases(g * (LANES * 9), g * LANES)
            return 0

        lax.fori_loop(0, n_groups, base_body, 0)
        compute_bases(xi_main, per_w)

        def bfrow(r, c):
            return plsc.bitcast(t_v[pl.ds(r + c, LANES)], jnp.bfloat16)

        def do_block(a0, ob, row0, blk):
            packed = []
            for u in range(blk):
                bv = base_v[pl.ds((a0 + u) * 4, LANES)]
                r0, r1, r2, r3 = bv[0], bv[1], bv[2], bv[3]
                packed.append([
                    (bfrow(r0, g * LANES) + bfrow(r1, g * LANES))
                    + (bfrow(r2, g * LANES) + bfrow(r3, g * LANES))
                    for g in range(EMB // (2 * LANES))
                ])
            for u in range(blk):
                for g, v in enumerate(packed[u]):
                    va, vb = plsc.unpack(v, format=plsc.PackFormat.INTERLEAVED)
                    ob[row0 + u, pl.ds(g * 2 * LANES, LANES)] = va
                    ob[row0 + u, pl.ds(g * 2 * LANES + LANES, LANES)] = vb

        def do_chunk(j, buf):
            ob = bufs[buf]

            def blk_body(ib, _):
                do_block(j * chunk + ib * BLK, ob, ib * BLK, BLK)
                return 0

            lax.fori_loop(0, chunk // BLK, blk_body, 0)
            pltpu.async_copy(
                ob,
                out_hbm.at[pl.ds(wid * per_w + j * chunk, chunk)],
                sems[buf],
            )

        def pair_body(jo, _):
            for b in range(2):
                j = jo * 2 + b
                @pl.when(jo > 0)
                def _wait():
                    pltpu.make_async_copy(
                        bufs[b],
                        out_hbm.at[pl.ds(0, chunk)],
                        sems[b],
                    ).wait()

                do_chunk(j, b)
            return 0

        def drain(b):
            pltpu.make_async_copy(
                bufs[b], out_hbm.at[pl.ds(0, chunk)], sems[b]
            ).wait()

        lax.fori_loop(0, n_chunks // 2, pair_body, 0)
        drain(0)
        drain(1)

        @pl.when(wid < tail_chunks)
        def _tail():
            do_block(per_w, o_v0, 0, 4)
            do_block(per_w + 4, o_v0, 4, 4)
            pltpu.sync_copy(
                o_v0.at[pl.ds(0, 8)],
                out_hbm.at[pl.ds(per_w * nw + wid * 8, 8)],
            )

    return launch


def kernel(x, W0, W1, W2, W3, W4, W5, W6, W7, W8):
    n = x.shape[0]
    t1 = (W2[:, None, :] + W5[None, :, :]).reshape(-1, EMB)
    t2 = (W3[:, None, :] + W4[None, :, :]).reshape(-1, EMB)
    t3 = (
        W1[:, None, None, None, :]
        + W6[None, :, None, None, :]
        + W7[None, None, :, None, :]
        + W8[None, None, None, :, :]
    ).reshape(-1, EMB)
    t = jnp.concatenate([W0, t1, t2, t3], axis=0)
    t = t.reshape(-1, 4, 2, LANES).transpose(0, 1, 3, 2)
    t = t.astype(jnp.bfloat16).reshape(-1, 2)
    t = jax.lax.bitcast_convert_type(t, jnp.int32).reshape(-1)

    xi = x.astype(jnp.int32).reshape(-1)
    xi = jnp.pad(xi, (0, 96))
    return _launcher(n)(xi, t)


# speedup vs baseline: 1.6141x; 1.4838x over previous
import functools

import jax
import jax.numpy as jnp
from jax import lax
from jax.experimental import pallas as pl
from jax.experimental.pallas import tpu as pltpu
from jax.experimental.pallas import tpu_sc as plsc

EMB = 128
LANES = 16
ROWW = EMB // 2
OFF1, OFF2, OFF3 = 119, 191, 311
TROWS = 407
BLK = 5


@functools.cache
def _launcher(n):
    nc, ns = 2, 16
    nw = nc * ns
    per_w = (n // nw) // 8 * 8
    n_tail = n - per_w * nw
    tail_chunks = n_tail // 8
    assert tail_chunks <= nw
    chunk = 120
    n_chunks = per_w // chunk
    assert n_chunks % 2 == 0 and chunk % BLK == 0
    col_v = per_w + LANES
    xi_size = 9 * col_v
    n_groups = per_w // LANES
    base_n = per_w + LANES
    mesh = plsc.VectorSubcoreMesh(
        core_axis_name="c", subcore_axis_name="s", num_cores=nc, num_subcores=ns
    )

    @functools.partial(
        pl.kernel,
        mesh=mesh,
        compiler_params=pltpu.CompilerParams(needs_layout_passes=False),
        out_type=jax.ShapeDtypeStruct((n, EMB), jnp.float32),
        scratch_types=[
            pltpu.VMEM((TROWS * ROWW,), jnp.int32),
            pltpu.VMEM((xi_size,), jnp.int32),
            pltpu.VMEM((4 * base_n,), jnp.int32),
            pltpu.VMEM((chunk, EMB), jnp.float32),
            pltpu.VMEM((chunk, EMB), jnp.float32),
            pltpu.SemaphoreType.DMA,
            pltpu.SemaphoreType.DMA,
        ],
    )
    def launch(xi_hbm, t_hbm, out_hbm, t_v, xi_v, base_v, o_v0, o_v1, sem0, sem1):
        wid = lax.axis_index("s") * nc + lax.axis_index("c")
        pltpu.sync_copy(t_hbm, t_v)
        m_off = pl.multiple_of(wid * per_w, 8)
        t_off = pl.multiple_of(
            per_w * nw + lax.min(wid, tail_chunks - 1) * 8, 8
        )
        for k in range(9):
            pltpu.sync_copy(
                xi_hbm.at[pl.ds(k * n + m_off, per_w)],
                xi_v.at[pl.ds(k * col_v, per_w)],
            )
            pltpu.sync_copy(
                xi_hbm.at[pl.ds(k * n + t_off, LANES)],
                xi_v.at[pl.ds(k * col_v + per_w, LANES)],
            )
        sems = (sem0, sem1)
        bufs = (o_v0, o_v1)
        lanes = jax.lax.iota(jnp.int32, LANES)

        def compute_bases(a0, atom0):
            xs = [xi_v[pl.ds(k * col_v + a0, LANES)] for k in range(9)]
            b0 = xs[0] * ROWW
            b1 = (xs[2] * 6 + xs[5] + OFF1) * ROWW
            b2 = (xs[3] * 10 + xs[4] + OFF2) * ROWW
            b3 = (((xs[1] * 6 + xs[6]) * 2 + xs[7]) * 2 + xs[8] + OFF3) * ROWW
            ov = (atom0 + lanes) * 4
            plsc.store_scatter(base_v, [ov], b0)
            plsc.store_scatter(base_v, [ov + 1], b1)
            plsc.store_scatter(base_v, [ov + 2], b2)
            plsc.store_scatter(base_v, [ov + 3], b3)

        def base_body(g, _):
            compute_bases(g * LANES, g * LANES)
            return 0

        lax.fori_loop(0, n_groups, base_body, 0)
        compute_bases(per_w, per_w)

        def bfrow(r, c):
            return plsc.bitcast(t_v[pl.ds(r + c, LANES)], jnp.bfloat16)

        def do_block(a0, ob, row0, blk):
            packed = []
            for u in range(blk):
                bv = base_v[pl.ds((a0 + u) * 4, LANES)]
                r0, r1, r2, r3 = bv[0], bv[1], bv[2], bv[3]
                packed.append([
                    (bfrow(r0, g * LANES) + bfrow(r1, g * LANES))
                    + (bfrow(r2, g * LANES) + bfrow(r3, g * LANES))
                    for g in range(EMB // (2 * LANES))
                ])
            for u in range(blk):
                for g, v in enumerate(packed[u]):
                    va, vb = plsc.unpack(v, format=plsc.PackFormat.INTERLEAVED)
                    ob[row0 + u, pl.ds(g * 2 * LANES, LANES)] = va
                    ob[row0 + u, pl.ds(g * 2 * LANES + LANES, LANES)] = vb

        def do_chunk(j, buf):
            ob = bufs[buf]

            def blk_body(ib, _):
                do_block(j * chunk + ib * BLK, ob, ib * BLK, BLK)
                return 0

            lax.fori_loop(0, chunk // BLK, blk_body, 0)
            pltpu.async_copy(
                ob,
                out_hbm.at[pl.ds(wid * per_w + j * chunk, chunk)],
                sems[buf],
            )

        def pair_body(jo, _):
            for b in range(2):
                j = jo * 2 + b
                @pl.when(jo > 0)
                def _wait():
                    pltpu.make_async_copy(
                        bufs[b],
                        out_hbm.at[pl.ds(0, chunk)],
                        sems[b],
                    ).wait()

                do_chunk(j, b)
            return 0

        def drain(b):
            pltpu.make_async_copy(
                bufs[b], out_hbm.at[pl.ds(0, chunk)], sems[b]
            ).wait()

        lax.fori_loop(0, n_chunks // 2, pair_body, 0)
        drain(0)
        drain(1)

        @pl.when(wid < tail_chunks)
        def _tail():
            do_block(per_w, o_v0, 0, 4)
            do_block(per_w + 4, o_v0, 4, 4)
            pltpu.sync_copy(
                o_v0.at[pl.ds(0, 8)],
                out_hbm.at[pl.ds(per_w * nw + wid * 8, 8)],
            )

    return launch


def kernel(x, W0, W1, W2, W3, W4, W5, W6, W7, W8):
    n = x.shape[0]
    t1 = (W2[:, None, :] + W5[None, :, :]).reshape(-1, EMB)
    t2 = (W3[:, None, :] + W4[None, :, :]).reshape(-1, EMB)
    t3 = (
        W1[:, None, None, None, :]
        + W6[None, :, None, None, :]
        + W7[None, None, :, None, :]
        + W8[None, None, None, :, :]
    ).reshape(-1, EMB)
    t = jnp.concatenate([W0, t1, t2, t3], axis=0)
    t = t.reshape(-1, 4, 2, LANES).transpose(0, 1, 3, 2)
    t = t.astype(jnp.bfloat16).reshape(-1, 2)
    t = jax.lax.bitcast_convert_type(t, jnp.int32).reshape(-1)

    xi = x.astype(jnp.int32).T.reshape(-1)
    return _launcher(n)(xi, t)
